# baseline (device time: 33345 ns/iter reference)
import jax
import jax.numpy as jnp
from jax import lax
from jax.experimental import pallas as pl
from jax.experimental.pallas import tpu as pltpu

N_DEV = 32
A_BITS = (0, 1, 3)
B_BITS = (2, 4)


def _mask(d, bits):
    return sum(((d >> j) & 1) << b for j, b in enumerate(bits))


def kernel(x, Wg, Wu, Wd):
    m, _ = x.shape
    d_out = Wd.shape[1]
    seg_a = m // 8
    seg_b = seg_a // 4

    def body(
        x_ref,
        wg_ref,
        wu_ref,
        wd_ref,
        out_ref,
        comm_ref,
        xb_ref,
        wgb_ref,
        wub_ref,
        wdb_ref,
        recv_a,
        recv_b,
        rsa_send,
        rsa_recv,
        rsb_send,
        rsb_recv,
        agb_send,
        agb_recv,
        aga_send,
        aga_recv,
    ):
        my_id = lax.axis_index("i")

        barrier = pltpu.get_barrier_semaphore()
        n_partners = 0
        for bits, radix in ((A_BITS, 8), (B_BITS, 4)):
            for d in range(1, radix):
                pl.semaphore_signal(
                    barrier,
                    inc=1,
                    device_id=(my_id ^ _mask(d, bits),),
                    device_id_type=pl.DeviceIdType.MESH,
                )
                n_partners += 1

        xb_ref[:] = x_ref[:].astype(jnp.bfloat16)
        wgb_ref[:] = wg_ref[:].astype(jnp.bfloat16)
        wub_ref[:] = wu_ref[:].astype(jnp.bfloat16)
        wdb_ref[:] = wd_ref[:].astype(jnp.bfloat16)

        qa = ((my_id >> A_BITS[0]) & 1) | (((my_id >> A_BITS[1]) & 1) << 1) | (
            ((my_id >> A_BITS[2]) & 1) << 2
        )
        qb = ((my_id >> B_BITS[0]) & 1) | (((my_id >> B_BITS[1]) & 1) << 1)

        def compute_chunk(r):
            rows = pl.ds(r * seg_a, seg_a)
            xc = xb_ref[rows, :]
            gate = jnp.dot(xc, wgb_ref[:], preferred_element_type=jnp.float32)
            up = jnp.dot(xc, wub_ref[:], preferred_element_type=jnp.float32)
            hc = (gate * (up * jax.nn.sigmoid(up))).astype(jnp.bfloat16)
            comm_ref[rows, :] = jnp.dot(
                hc, wdb_ref[:], preferred_element_type=jnp.float32
            ).astype(jnp.bfloat16)

        rdmas = []
        for d in range(7, 0, -1):
            pq = qa ^ d
            compute_chunk(pq)
            if d == 7:
                pl.semaphore_wait(barrier, n_partners)
            rdma = pltpu.make_async_remote_copy(
                src_ref=comm_ref.at[pl.ds(pq * seg_a, seg_a), :],
                dst_ref=recv_a.at[d - 1],
                send_sem=rsa_send.at[d - 1],
                recv_sem=rsa_recv.at[d - 1],
                device_id=(my_id ^ _mask(d, A_BITS),),
                device_id_type=pl.DeviceIdType.MESH,
            )
            rdma.start()
            rdmas.append(rdma)
        compute_chunk(qa)
        for rdma in rdmas:
            rdma.wait()
        lo = qa * seg_a
        acc = comm_ref[pl.ds(lo, seg_a), :].astype(jnp.float32)
        for d in range(1, 8):
            acc = acc + recv_a[d - 1, :, :].astype(jnp.float32)
        comm_ref[pl.ds(lo, seg_a), :] = acc.astype(jnp.bfloat16)

        rdmas = []
        for d in range(1, 4):
            pq = qb ^ d
            rdma = pltpu.make_async_remote_copy(
                src_ref=comm_ref.at[pl.ds(lo + pq * seg_b, seg_b), :],
                dst_ref=recv_b.at[d - 1],
                send_sem=rsb_send.at[d - 1],
                recv_sem=rsb_recv.at[d - 1],
                device_id=(my_id ^ _mask(d, B_BITS),),
                device_id_type=pl.DeviceIdType.MESH,
            )
            rdma.start()
            rdmas.append(rdma)
        for rdma in rdmas:
            rdma.wait()
        lo = lo + qb * seg_b
        acc = comm_ref[pl.ds(lo, seg_b), :].astype(jnp.float32)
        for d in range(1, 4):
            acc = acc + recv_b[d - 1, :, :].astype(jnp.float32)
        comm_ref[pl.ds(lo, seg_b), :] = acc.astype(jnp.bfloat16)

        rdmas = []
        for d in range(1, 4):
            rdma = pltpu.make_async_remote_copy(
                src_ref=comm_ref.at[pl.ds(lo, seg_b), :],
                dst_ref=comm_ref.at[pl.ds(lo, seg_b), :],
                send_sem=agb_send.at[d - 1],
                recv_sem=agb_recv.at[d - 1],
                device_id=(my_id ^ _mask(d, B_BITS),),
                device_id_type=pl.DeviceIdType.MESH,
            )
            rdma.start()
            rdmas.append(rdma)
        for rdma in rdmas:
            rdma.wait()
        lo = lo - qb * seg_b

        rdmas = []
        for d in range(1, 8):
            rdma = pltpu.make_async_remote_copy(
                src_ref=comm_ref.at[pl.ds(lo, seg_a), :],
                dst_ref=comm_ref.at[pl.ds(lo, seg_a), :],
                send_sem=aga_send.at[d - 1],
                recv_sem=aga_recv.at[d - 1],
                device_id=(my_id ^ _mask(d, A_BITS),),
                device_id_type=pl.DeviceIdType.MESH,
            )
            rdma.start()
            rdmas.append(rdma)
        out_ref[pl.ds(lo, seg_a), :] = comm_ref[pl.ds(lo, seg_a), :].astype(
            jnp.float32
        )
        for d, rdma in zip(range(1, 8), rdmas):
            rdma.wait()
            rows = pl.ds((qa ^ d) * seg_a, seg_a)
            out_ref[rows, :] = comm_ref[rows, :].astype(jnp.float32)

    return pl.pallas_call(
        body,
        out_shape=jax.ShapeDtypeStruct((m, d_out), jnp.float32),
        in_specs=[pl.BlockSpec(memory_space=pltpu.VMEM)] * 4,
        out_specs=pl.BlockSpec(memory_space=pltpu.VMEM),
        scratch_shapes=[
            pltpu.VMEM((m, d_out), jnp.bfloat16),
            pltpu.VMEM(x.shape, jnp.bfloat16),
            pltpu.VMEM(Wg.shape, jnp.bfloat16),
            pltpu.VMEM(Wu.shape, jnp.bfloat16),
            pltpu.VMEM(Wd.shape, jnp.bfloat16),
            pltpu.VMEM((7, seg_a, d_out), jnp.bfloat16),
            pltpu.VMEM((3, seg_b, d_out), jnp.bfloat16),
            pltpu.SemaphoreType.DMA((7,)),
            pltpu.SemaphoreType.DMA((7,)),
            pltpu.SemaphoreType.DMA((3,)),
            pltpu.SemaphoreType.DMA((3,)),
            pltpu.SemaphoreType.DMA((3,)),
            pltpu.SemaphoreType.DMA((3,)),
            pltpu.SemaphoreType.DMA((7,)),
            pltpu.SemaphoreType.DMA((7,)),
        ],
        compiler_params=pltpu.CompilerParams(collective_id=0),
    )(x, Wg, Wu, Wd)


# device time: 31451 ns/iter; 1.0602x vs baseline; 1.0602x over previous
import jax
import jax.numpy as jnp
from jax import lax
from jax.experimental import pallas as pl
from jax.experimental.pallas import tpu as pltpu

N_DEV = 32
A_BITS = (0, 1, 3)
B_BITS = (2, 4)


def _mask(d, bits):
    return sum(((d >> j) & 1) << b for j, b in enumerate(bits))


def kernel(x, Wg, Wu, Wd):
    m, _ = x.shape
    d_out = Wd.shape[1]
    seg_a = m // 8
    seg_b = seg_a // 4

    def body(
        x_ref,
        wg_ref,
        wu_ref,
        wd_ref,
        out_ref,
        comm_ref,
        xb_ref,
        wgb_ref,
        wub_ref,
        wdb_ref,
        recv_a,
        recv_b,
        rsa_send,
        rsa_recv,
        rsb_send,
        rsb_recv,
        agb_send,
        agb_recv,
        aga_send,
        aga_recv,
    ):
        my_id = lax.axis_index("i")

        barrier = pltpu.get_barrier_semaphore()
        n_partners = 0
        for bits, radix in ((A_BITS, 8), (B_BITS, 4)):
            for d in range(1, radix):
                pl.semaphore_signal(
                    barrier,
                    inc=1,
                    device_id=(my_id ^ _mask(d, bits),),
                    device_id_type=pl.DeviceIdType.MESH,
                )
                n_partners += 1

        xb_ref[:] = x_ref[:].astype(jnp.bfloat16)
        wgb_ref[:] = wg_ref[:].astype(jnp.bfloat16)
        wub_ref[:] = wu_ref[:].astype(jnp.bfloat16)
        wdb_ref[:] = wd_ref[:].astype(jnp.bfloat16)

        qa = ((my_id >> A_BITS[0]) & 1) | (((my_id >> A_BITS[1]) & 1) << 1) | (
            ((my_id >> A_BITS[2]) & 1) << 2
        )
        qb = ((my_id >> B_BITS[0]) & 1) | (((my_id >> B_BITS[1]) & 1) << 1)

        def compute_half(h):
            rows = pl.ds(h * (m // 2), m // 2)
            xc = xb_ref[rows, :]
            gate = jnp.dot(xc, wgb_ref[:], preferred_element_type=jnp.float32)
            up = jnp.dot(xc, wub_ref[:], preferred_element_type=jnp.float32)
            hc = (gate * (up * jax.nn.sigmoid(up))).astype(jnp.bfloat16)
            comm_ref[rows, :] = jnp.dot(
                hc, wdb_ref[:], preferred_element_type=jnp.float32
            ).astype(jnp.bfloat16)

        def start_rsa(d):
            pq = qa ^ d
            rdma = pltpu.make_async_remote_copy(
                src_ref=comm_ref.at[pl.ds(pq * seg_a, seg_a), :],
                dst_ref=recv_a.at[d - 1],
                send_sem=rsa_send.at[d - 1],
                recv_sem=rsa_recv.at[d - 1],
                device_id=(my_id ^ _mask(d, A_BITS),),
                device_id_type=pl.DeviceIdType.MESH,
            )
            rdma.start()
            return rdma

        my_half = qa >> 2
        compute_half(1 - my_half)
        pl.semaphore_wait(barrier, n_partners)
        rdmas = [start_rsa(d) for d in range(4, 8)]
        compute_half(my_half)
        rdmas += [start_rsa(d) for d in range(1, 4)]
        for rdma in rdmas:
            rdma.wait()
        lo = qa * seg_a
        acc = comm_ref[pl.ds(lo, seg_a), :].astype(jnp.float32)
        for d in range(1, 8):
            acc = acc + recv_a[d - 1, :, :].astype(jnp.float32)
        comm_ref[pl.ds(lo, seg_a), :] = acc.astype(jnp.bfloat16)

        rdmas = []
        for d in range(1, 4):
            pq = qb ^ d
            rdma = pltpu.make_async_remote_copy(
                src_ref=comm_ref.at[pl.ds(lo + pq * seg_b, seg_b), :],
                dst_ref=recv_b.at[d - 1],
                send_sem=rsb_send.at[d - 1],
                recv_sem=rsb_recv.at[d - 1],
                device_id=(my_id ^ _mask(d, B_BITS),),
                device_id_type=pl.DeviceIdType.MESH,
            )
            rdma.start()
            rdmas.append(rdma)
        for rdma in rdmas:
            rdma.wait()
        lo = lo + qb * seg_b
        acc = comm_ref[pl.ds(lo, seg_b), :].astype(jnp.float32)
        for d in range(1, 4):
            acc = acc + recv_b[d - 1, :, :].astype(jnp.float32)
        comm_ref[pl.ds(lo, seg_b), :] = acc.astype(jnp.bfloat16)

        rdmas = []
        for d in range(1, 4):
            rdma = pltpu.make_async_remote_copy(
                src_ref=comm_ref.at[pl.ds(lo, seg_b), :],
                dst_ref=comm_ref.at[pl.ds(lo, seg_b), :],
                send_sem=agb_send.at[d - 1],
                recv_sem=agb_recv.at[d - 1],
                device_id=(my_id ^ _mask(d, B_BITS),),
                device_id_type=pl.DeviceIdType.MESH,
            )
            rdma.start()
            rdmas.append(rdma)
        for rdma in rdmas:
            rdma.wait()
        lo = lo - qb * seg_b

        rdmas = []
        for d in range(1, 8):
            rdma = pltpu.make_async_remote_copy(
                src_ref=comm_ref.at[pl.ds(lo, seg_a), :],
                dst_ref=comm_ref.at[pl.ds(lo, seg_a), :],
                send_sem=aga_send.at[d - 1],
                recv_sem=aga_recv.at[d - 1],
                device_id=(my_id ^ _mask(d, A_BITS),),
                device_id_type=pl.DeviceIdType.MESH,
            )
            rdma.start()
            rdmas.append(rdma)
        out_ref[pl.ds(lo, seg_a), :] = comm_ref[pl.ds(lo, seg_a), :].astype(
            jnp.float32
        )
        for d, rdma in zip(range(1, 8), rdmas):
            rdma.wait()
            rows = pl.ds((qa ^ d) * seg_a, seg_a)
            out_ref[rows, :] = comm_ref[rows, :].astype(jnp.float32)

    return pl.pallas_call(
        body,
        out_shape=jax.ShapeDtypeStruct((m, d_out), jnp.float32),
        in_specs=[pl.BlockSpec(memory_space=pltpu.VMEM)] * 4,
        out_specs=pl.BlockSpec(memory_space=pltpu.VMEM),
        scratch_shapes=[
            pltpu.VMEM((m, d_out), jnp.bfloat16),
            pltpu.VMEM(x.shape, jnp.bfloat16),
            pltpu.VMEM(Wg.shape, jnp.bfloat16),
            pltpu.VMEM(Wu.shape, jnp.bfloat16),
            pltpu.VMEM(Wd.shape, jnp.bfloat16),
            pltpu.VMEM((7, seg_a, d_out), jnp.bfloat16),
            pltpu.VMEM((3, seg_b, d_out), jnp.bfloat16),
            pltpu.SemaphoreType.DMA((7,)),
            pltpu.SemaphoreType.DMA((7,)),
            pltpu.SemaphoreType.DMA((3,)),
            pltpu.SemaphoreType.DMA((3,)),
            pltpu.SemaphoreType.DMA((3,)),
            pltpu.SemaphoreType.DMA((3,)),
            pltpu.SemaphoreType.DMA((7,)),
            pltpu.SemaphoreType.DMA((7,)),
        ],
        compiler_params=pltpu.CompilerParams(collective_id=0),
    )(x, Wg, Wu, Wd)


# device time: 31132 ns/iter; 1.0711x vs baseline; 1.0102x over previous
import jax
import jax.numpy as jnp
from jax import lax
from jax.experimental import pallas as pl
from jax.experimental.pallas import tpu as pltpu

N_DEV = 32
A_BITS = (0, 1, 3)
B_BITS = (2, 4)


def _mask(d, bits):
    return sum(((d >> j) & 1) << b for j, b in enumerate(bits))


def kernel(x, Wg, Wu, Wd):
    m, _ = x.shape
    d_out = Wd.shape[1]
    seg_a = m // 8
    seg_b = seg_a // 4

    def body(
        x_ref,
        wg_ref,
        wu_ref,
        wd_ref,
        out_ref,
        comm_ref,
        xb_ref,
        wgb_ref,
        wub_ref,
        wdb_ref,
        recv_a,
        recv_b,
        rsa_send,
        rsa_recv,
        rsb_send,
        rsb_recv,
        agb_send,
        agb_recv,
        aga_send,
        aga_recv,
    ):
        my_id = lax.axis_index("i")

        barrier = pltpu.get_barrier_semaphore()
        n_partners = 0
        for bits, radix in ((A_BITS, 8), (B_BITS, 4)):
            for d in range(1, radix):
                pl.semaphore_signal(
                    barrier,
                    inc=1,
                    device_id=(my_id ^ _mask(d, bits),),
                    device_id_type=pl.DeviceIdType.MESH,
                )
                n_partners += 1

        xb_ref[:] = x_ref[:].astype(jnp.bfloat16)
        wgb_ref[:] = wg_ref[:].astype(jnp.bfloat16)
        wub_ref[:] = wu_ref[:].astype(jnp.bfloat16)
        wdb_ref[:] = wd_ref[:].astype(jnp.bfloat16)

        qa = ((my_id >> A_BITS[0]) & 1) | (((my_id >> A_BITS[1]) & 1) << 1) | (
            ((my_id >> A_BITS[2]) & 1) << 2
        )
        qb = ((my_id >> B_BITS[0]) & 1) | (((my_id >> B_BITS[1]) & 1) << 1)

        def start_rsa(d):
            pq = qa ^ d
            rdma = pltpu.make_async_remote_copy(
                src_ref=comm_ref.at[pl.ds(pq * seg_a, seg_a), :],
                dst_ref=recv_a.at[d - 1],
                send_sem=rsa_send.at[d - 1],
                recv_sem=rsa_recv.at[d - 1],
                device_id=(my_id ^ _mask(d, A_BITS),),
                device_id_type=pl.DeviceIdType.MESH,
            )
            rdma.start()
            return rdma

        gate = jnp.dot(xb_ref[:], wgb_ref[:], preferred_element_type=jnp.float32)
        up = jnp.dot(xb_ref[:], wub_ref[:], preferred_element_type=jnp.float32)
        hb = (gate * (up * jax.nn.sigmoid(up))).astype(jnp.bfloat16)
        comm_ref[:] = jnp.dot(
            hb, wdb_ref[:], preferred_element_type=jnp.float32
        ).astype(jnp.bfloat16)

        pl.semaphore_wait(barrier, n_partners)
        rdmas = [start_rsa(d) for d in range(1, 8)]
        for rdma in rdmas:
            rdma.wait()
        lo = qa * seg_a
        acc = comm_ref[pl.ds(lo, seg_a), :].astype(jnp.float32)
        for d in range(1, 8):
            acc = acc + recv_a[d - 1, :, :].astype(jnp.float32)
        comm_ref[pl.ds(lo, seg_a), :] = acc.astype(jnp.bfloat16)

        rdmas = []
        for d in range(1, 4):
            pq = qb ^ d
            rdma = pltpu.make_async_remote_copy(
                src_ref=comm_ref.at[pl.ds(lo + pq * seg_b, seg_b), :],
                dst_ref=recv_b.at[d - 1],
                send_sem=rsb_send.at[d - 1],
                recv_sem=rsb_recv.at[d - 1],
                device_id=(my_id ^ _mask(d, B_BITS),),
                device_id_type=pl.DeviceIdType.MESH,
            )
            rdma.start()
            rdmas.append(rdma)
        for rdma in rdmas:
            rdma.wait()
        lo = lo + qb * seg_b
        acc = comm_ref[pl.ds(lo, seg_b), :].astype(jnp.float32)
        for d in range(1, 4):
            acc = acc + recv_b[d - 1, :, :].astype(jnp.float32)
        comm_ref[pl.ds(lo, seg_b), :] = acc.astype(jnp.bfloat16)

        rdmas = []
        for d in range(1, 4):
            rdma = pltpu.make_async_remote_copy(
                src_ref=comm_ref.at[pl.ds(lo, seg_b), :],
                dst_ref=comm_ref.at[pl.ds(lo, seg_b), :],
                send_sem=agb_send.at[d - 1],
                recv_sem=agb_recv.at[d - 1],
                device_id=(my_id ^ _mask(d, B_BITS),),
                device_id_type=pl.DeviceIdType.MESH,
            )
            rdma.start()
            rdmas.append(rdma)
        for rdma in rdmas:
            rdma.wait()
        lo = lo - qb * seg_b

        rdmas = []
        for d in range(1, 8):
            rdma = pltpu.make_async_remote_copy(
                src_ref=comm_ref.at[pl.ds(lo, seg_a), :],
                dst_ref=comm_ref.at[pl.ds(lo, seg_a), :],
                send_sem=aga_send.at[d - 1],
                recv_sem=aga_recv.at[d - 1],
                device_id=(my_id ^ _mask(d, A_BITS),),
                device_id_type=pl.DeviceIdType.MESH,
            )
            rdma.start()
            rdmas.append(rdma)
        out_ref[pl.ds(lo, seg_a), :] = comm_ref[pl.ds(lo, seg_a), :].astype(
            jnp.float32
        )
        for d, rdma in zip(range(1, 8), rdmas):
            rdma.wait()
            rows = pl.ds((qa ^ d) * seg_a, seg_a)
            out_ref[rows, :] = comm_ref[rows, :].astype(jnp.float32)

    return pl.pallas_call(
        body,
        out_shape=jax.ShapeDtypeStruct((m, d_out), jnp.float32),
        in_specs=[pl.BlockSpec(memory_space=pltpu.VMEM)] * 4,
        out_specs=pl.BlockSpec(memory_space=pltpu.VMEM),
        scratch_shapes=[
            pltpu.VMEM((m, d_out), jnp.bfloat16),
            pltpu.VMEM(x.shape, jnp.bfloat16),
            pltpu.VMEM(Wg.shape, jnp.bfloat16),
            pltpu.VMEM(Wu.shape, jnp.bfloat16),
            pltpu.VMEM(Wd.shape, jnp.bfloat16),
            pltpu.VMEM((7, seg_a, d_out), jnp.bfloat16),
            pltpu.VMEM((3, seg_b, d_out), jnp.bfloat16),
            pltpu.SemaphoreType.DMA((7,)),
            pltpu.SemaphoreType.DMA((7,)),
            pltpu.SemaphoreType.DMA((3,)),
            pltpu.SemaphoreType.DMA((3,)),
            pltpu.SemaphoreType.DMA((3,)),
            pltpu.SemaphoreType.DMA((3,)),
            pltpu.SemaphoreType.DMA((7,)),
            pltpu.SemaphoreType.DMA((7,)),
        ],
        compiler_params=pltpu.CompilerParams(collective_id=0),
    )(x, Wg, Wu, Wd)


# device time: 29333 ns/iter; 1.1368x vs baseline; 1.0613x over previous
import jax
import jax.numpy as jnp
from jax import lax
from jax.experimental import pallas as pl
from jax.experimental.pallas import tpu as pltpu

N_DEV = 32
A_BITS = (0, 1, 3)
B_BITS = (2, 4)


def _mask(d, bits):
    return sum(((d >> j) & 1) << b for j, b in enumerate(bits))


def kernel(x, Wg, Wu, Wd):
    m, _ = x.shape
    d_out = Wd.shape[1]
    seg_a = m // 8
    seg_b = seg_a // 4

    def body(
        x_ref,
        wg_ref,
        wu_ref,
        wd_ref,
        out_ref,
        comm_ref,
        recv_a,
        recv_b,
        rsa_send,
        rsa_recv,
        rsb_send,
        rsb_recv,
        agb_send,
        agb_recv,
        aga_send,
        aga_recv,
    ):
        my_id = lax.axis_index("i")

        barrier = pltpu.get_barrier_semaphore()
        n_partners = 0
        for bits, radix in ((A_BITS, 8), (B_BITS, 4)):
            for d in range(1, radix):
                pl.semaphore_signal(
                    barrier,
                    inc=1,
                    device_id=(my_id ^ _mask(d, bits),),
                    device_id_type=pl.DeviceIdType.MESH,
                )
                n_partners += 1

        xb = x_ref[:].astype(jnp.bfloat16)
        gate = jnp.dot(
            xb, wg_ref[:].astype(jnp.bfloat16), preferred_element_type=jnp.float32
        )
        up = jnp.dot(
            xb, wu_ref[:].astype(jnp.bfloat16), preferred_element_type=jnp.float32
        )
        h = (gate * (up * jax.nn.sigmoid(up))).astype(jnp.bfloat16)
        comm_ref[:] = jnp.dot(
            h, wd_ref[:].astype(jnp.bfloat16), preferred_element_type=jnp.float32
        ).astype(jnp.bfloat16)

        pl.semaphore_wait(barrier, n_partners)

        qa = ((my_id >> A_BITS[0]) & 1) | (((my_id >> A_BITS[1]) & 1) << 1) | (
            ((my_id >> A_BITS[2]) & 1) << 2
        )
        rdmas = []
        for d in range(1, 8):
            pq = qa ^ d
            rdma = pltpu.make_async_remote_copy(
                src_ref=comm_ref.at[pl.ds(pq * seg_a, seg_a), :],
                dst_ref=recv_a.at[d - 1],
                send_sem=rsa_send.at[d - 1],
                recv_sem=rsa_recv.at[d - 1],
                device_id=(my_id ^ _mask(d, A_BITS),),
                device_id_type=pl.DeviceIdType.MESH,
            )
            rdma.start()
            rdmas.append(rdma)
        for rdma in rdmas:
            rdma.wait()
        lo = qa * seg_a
        acc = comm_ref[pl.ds(lo, seg_a), :].astype(jnp.float32)
        for d in range(1, 8):
            acc = acc + recv_a[d - 1, :, :].astype(jnp.float32)
        comm_ref[pl.ds(lo, seg_a), :] = acc.astype(jnp.bfloat16)

        qb = ((my_id >> B_BITS[0]) & 1) | (((my_id >> B_BITS[1]) & 1) << 1)
        rdmas = []
        for d in range(1, 4):
            pq = qb ^ d
            rdma = pltpu.make_async_remote_copy(
                src_ref=comm_ref.at[pl.ds(lo + pq * seg_b, seg_b), :],
                dst_ref=recv_b.at[d - 1],
                send_sem=rsb_send.at[d - 1],
                recv_sem=rsb_recv.at[d - 1],
                device_id=(my_id ^ _mask(d, B_BITS),),
                device_id_type=pl.DeviceIdType.MESH,
            )
            rdma.start()
            rdmas.append(rdma)
        for rdma in rdmas:
            rdma.wait()
        lo = lo + qb * seg_b
        acc = comm_ref[pl.ds(lo, seg_b), :].astype(jnp.float32)
        for d in range(1, 4):
            acc = acc + recv_b[d - 1, :, :].astype(jnp.float32)
        comm_ref[pl.ds(lo, seg_b), :] = acc.astype(jnp.bfloat16)

        rdmas = []
        for d in range(1, 4):
            rdma = pltpu.make_async_remote_copy(
                src_ref=comm_ref.at[pl.ds(lo, seg_b), :],
                dst_ref=comm_ref.at[pl.ds(lo, seg_b), :],
                send_sem=agb_send.at[d - 1],
                recv_sem=agb_recv.at[d - 1],
                device_id=(my_id ^ _mask(d, B_BITS),),
                device_id_type=pl.DeviceIdType.MESH,
            )
            rdma.start()
            rdmas.append(rdma)
        for rdma in rdmas:
            rdma.wait()
        lo = lo - qb * seg_b

        rdmas = []
        for d in range(1, 8):
            rdma = pltpu.make_async_remote_copy(
                src_ref=comm_ref.at[pl.ds(lo, seg_a), :],
                dst_ref=comm_ref.at[pl.ds(lo, seg_a), :],
                send_sem=aga_send.at[d - 1],
                recv_sem=aga_recv.at[d - 1],
                device_id=(my_id ^ _mask(d, A_BITS),),
                device_id_type=pl.DeviceIdType.MESH,
            )
            rdma.start()
            rdmas.append(rdma)
        for rdma in rdmas:
            rdma.wait()

        out_ref[:] = comm_ref[:].astype(jnp.float32)

    return pl.pallas_call(
        body,
        out_shape=jax.ShapeDtypeStruct((m, d_out), jnp.float32),
        in_specs=[pl.BlockSpec(memory_space=pltpu.VMEM)] * 4,
        out_specs=pl.BlockSpec(memory_space=pltpu.VMEM),
        scratch_shapes=[
            pltpu.VMEM((m, d_out), jnp.bfloat16),
            pltpu.VMEM((7, seg_a, d_out), jnp.bfloat16),
            pltpu.VMEM((3, seg_b, d_out), jnp.bfloat16),
            pltpu.SemaphoreType.DMA((7,)),
            pltpu.SemaphoreType.DMA((7,)),
            pltpu.SemaphoreType.DMA((3,)),
            pltpu.SemaphoreType.DMA((3,)),
            pltpu.SemaphoreType.DMA((3,)),
            pltpu.SemaphoreType.DMA((3,)),
            pltpu.SemaphoreType.DMA((7,)),
            pltpu.SemaphoreType.DMA((7,)),
        ],
        compiler_params=pltpu.CompilerParams(collective_id=0),
    )(x, Wg, Wu, Wd)
